# Initial kernel scaffold; baseline (speedup 1.0000x reference)
import jax
import jax.numpy as jnp
from jax.experimental import pallas as pl


def kernel(x, edge_index, edge_weight, W1, b1, W_out, b_out, W_root):
    raise NotImplementedError("write your pallas kernel here")


# speedup vs baseline: 16.9929x; 16.9929x over previous
import functools

import jax
import jax.numpy as jnp
from jax import lax
from jax.experimental import pallas as pl
from jax.experimental.pallas import tpu as pltpu
from jax.experimental.pallas import tpu_sc as plsc

N = 10000
E = 320000
D = 128
H = 128
EB = E // 128
NPAD = 10240
PART = NPAD // 16

_MESH = plsc.VectorSubcoreMesh(core_axis_name="c", subcore_axis_name="s")
_SC_PARAMS = pltpu.CompilerParams(needs_layout_passes=False)

_BASE = EB // 32
_REM = EB - _BASE * 32


def _worker_id():
    return lax.axis_index("s") * 2 + lax.axis_index("c")


def _block_range(w):
    start = _BASE * w + jnp.minimum(w, _REM)
    n = jnp.where(w < _REM, _BASE + 1, _BASE)
    return start, n


def _splat(e):
    return jnp.full((16,), e, dtype=jnp.int32)


@functools.partial(
    pl.kernel,
    mesh=_MESH,
    compiler_params=_SC_PARAMS,
    out_type=jax.ShapeDtypeStruct((2, NPAD), jnp.float32),
    scratch_types=[
        pltpu.VMEM((1, 128), jnp.int32),
        pltpu.VMEM((128,), jnp.float32),
        pltpu.VMEM((PART,), jnp.float32),
        pltpu.VMEM_SHARED((NPAD,), jnp.float32),
    ],
)
def _sc_deg1(col_hbm, ew_hbm, out_hbm, cidx, ewb, zrow, acc):
    cid = lax.axis_index("c")
    sid = lax.axis_index("s")
    w = _worker_id()

    def zbody(i, _):
        zrow[pl.ds(i * 16, 16)] = jnp.zeros((16,), jnp.float32)
        return 0
    lax.fori_loop(0, PART // 16, zbody, 0)
    pltpu.sync_copy(zrow, acc.at[pl.ds(sid * PART, PART)])
    plsc.subcore_barrier()

    start, nblk = _block_range(w)

    def body(i, _):
        blk = start + i
        pltpu.sync_copy(col_hbm.at[blk], cidx.at[0])
        pltpu.sync_copy(ew_hbm.at[blk], ewb)
        pltpu.sync_copy(ewb, acc.at[cidx.at[0]], add=True)
        return 0
    lax.fori_loop(0, nblk, body, 0)

    plsc.subcore_barrier()
    pltpu.sync_copy(acc.at[pl.ds(sid * PART, PART)],
                    out_hbm.at[cid, pl.ds(sid * PART, PART)])


@functools.partial(
    pl.kernel,
    mesh=_MESH,
    compiler_params=_SC_PARAMS,
    out_type=jax.ShapeDtypeStruct((2, NPAD, 128), jnp.float32),
    scratch_types=[
        pltpu.VMEM((1, 128), jnp.int32),
        pltpu.VMEM((1, 128), jnp.int32),
        pltpu.VMEM((128,), jnp.float32),
        pltpu.VMEM((128, 128), jnp.float32),
        pltpu.VMEM((128, 128), jnp.float32),
        pltpu.VMEM_SHARED((NPAD, 128), jnp.float32),
        pltpu.SemaphoreType.DMA,
    ],
)
def _sc_spmm(row_hbm, col_hbm, ew_hbm, hp_hbm, out_hbm,
             ridx, cidx, ewb, rows, zblk, acc, gsem):
    cid = lax.axis_index("c")
    sid = lax.axis_index("s")
    w = _worker_id()

    def zbody(i, _):
        for j in range(8):
            zblk[i, pl.ds(j * 16, 16)] = jnp.zeros((16,), jnp.float32)
        return 0
    lax.fori_loop(0, 128, zbody, 0)
    for k in range(PART // 128):
        pltpu.sync_copy(zblk, acc.at[pl.ds(sid * PART + k * 128, 128)])
    plsc.subcore_barrier()

    start, nblk = _block_range(w)

    def body(i, _):
        blk = start + i
        pltpu.sync_copy(row_hbm.at[blk], ridx.at[0])
        pltpu.sync_copy(col_hbm.at[blk], cidx.at[0])
        pltpu.sync_copy(ew_hbm.at[blk], ewb)
        pltpu.async_copy(hp_hbm.at[ridx.at[0]], rows, gsem).wait()

        def ebody(e, _):
            w16 = plsc.load_gather(ewb, [_splat(e)])
            for j in range(8):
                sl = pl.ds(j * 16, 16)
                rows[e, sl] = rows[e, sl] * w16
            return 0
        lax.fori_loop(0, 128, ebody, 0)
        pltpu.sync_copy(rows, acc.at[cidx.at[0]], add=True)
        return 0
    lax.fori_loop(0, nblk, body, 0)

    plsc.subcore_barrier()
    pltpu.sync_copy(acc.at[pl.ds(sid * PART, PART)],
                    out_hbm.at[cid, pl.ds(sid * PART, PART)])


@functools.partial(
    pl.kernel,
    mesh=_MESH,
    compiler_params=_SC_PARAMS,
    out_type=[jax.ShapeDtypeStruct((2, NPAD), jnp.float32),
              jax.ShapeDtypeStruct((2, NPAD), jnp.float32)],
    scratch_types=[
        pltpu.VMEM((1, 128), jnp.int32),
        pltpu.VMEM((1, 128), jnp.int32),
        pltpu.VMEM((128,), jnp.float32),
        pltpu.VMEM((128,), jnp.float32),
        pltpu.VMEM((N,), jnp.float32),
        pltpu.VMEM((PART,), jnp.float32),
        pltpu.VMEM_SHARED((NPAD,), jnp.float32),
        pltpu.VMEM_SHARED((NPAD,), jnp.float32),
    ],
)
def _sc_layer2(row_hbm, col_hbm, s_hbm, tout_hbm, dout_hbm,
               ridx, cidx, updb, maskb, sv, zrow, tacc, dacc):
    cid = lax.axis_index("c")
    sid = lax.axis_index("s")
    w = _worker_id()

    def zbody(i, _):
        zrow[pl.ds(i * 16, 16)] = jnp.zeros((16,), jnp.float32)
        return 0
    lax.fori_loop(0, PART // 16, zbody, 0)
    pltpu.sync_copy(zrow, tacc.at[pl.ds(sid * PART, PART)])
    pltpu.sync_copy(zrow, dacc.at[pl.ds(sid * PART, PART)])
    pltpu.sync_copy(s_hbm, sv)
    plsc.subcore_barrier()

    start, nblk = _block_range(w)

    def body(i, _):
        blk = start + i
        pltpu.sync_copy(row_hbm.at[blk], ridx.at[0])
        pltpu.sync_copy(col_hbm.at[blk], cidx.at[0])

        def gbody(k, _):
            sl = pl.ds(k * 16, 16)
            r16 = ridx[0, sl]
            c16 = cidx[0, sl]
            sval = plsc.load_gather(sv, [r16])
            m = r16 != c16
            updb[sl] = jnp.where(m, sval, 0.0)
            maskb[sl] = jnp.where(m, 1.0, 0.0)
            return 0
        lax.fori_loop(0, 8, gbody, 0)
        pltpu.sync_copy(updb, tacc.at[cidx.at[0]], add=True)
        pltpu.sync_copy(maskb, dacc.at[cidx.at[0]], add=True)
        return 0
    lax.fori_loop(0, nblk, body, 0)

    plsc.subcore_barrier()
    pltpu.sync_copy(tacc.at[pl.ds(sid * PART, PART)],
                    tout_hbm.at[cid, pl.ds(sid * PART, PART)])
    pltpu.sync_copy(dacc.at[pl.ds(sid * PART, PART)],
                    dout_hbm.at[cid, pl.ds(sid * PART, PART)])


def _tc_mm_body(x_ref, w_ref, o_ref):
    o_ref[...] = jnp.dot(x_ref[...], w_ref[...],
                         preferred_element_type=jnp.float32)


def _tc_prep_body(dp_ref, h_ref, dis_ref, hp_ref):
    deg = dp_ref[0, :N, :] + dp_ref[1, :N, :] + 1.0
    dis = jnp.where(deg > 0, lax.rsqrt(jnp.maximum(deg, 1e-12)), 0.0)
    dis_ref[...] = dis
    hp_ref[...] = h_ref[...] * dis


def _tc_mid_body(S_ref, hp_ref, dis_ref, b1_ref, wc_ref, o_ref):
    pre = dis_ref[...] * (S_ref[0, :N, :] + S_ref[1, :N, :] + hp_ref[...]) \
        + b1_ref[...]
    hr = jnp.maximum(pre, 0.0)
    o_ref[...] = jnp.dot(hr, wc_ref[...], preferred_element_type=jnp.float32)


def _tc_final_body(tp_ref, dp_ref, sr_ref, bo_ref, o_ref):
    t = tp_ref[0, :N, :] + tp_ref[1, :N, :]
    d2 = jnp.maximum(dp_ref[0, :N, :] + dp_ref[1, :N, :] + 1.0, 1.0)
    s = sr_ref[:, 0:1]
    r = sr_ref[:, 1:2]
    o_ref[...] = (t + s) / d2 + bo_ref[0, 0] + r


def kernel(x, edge_index, edge_weight, W1, b1, W_out, b_out, W_root):
    row2d = edge_index[0].reshape(EB, 128)
    col2d = edge_index[1].reshape(EB, 128)
    ew2d = edge_weight.reshape(EB, 128)

    deg1p = _sc_deg1(col2d, ew2d)

    h = pl.pallas_call(
        _tc_mm_body,
        out_shape=jax.ShapeDtypeStruct((N, H), jnp.float32),
    )(x, W1)

    dis, hp = pl.pallas_call(
        _tc_prep_body,
        out_shape=[jax.ShapeDtypeStruct((N, 1), jnp.float32),
                   jax.ShapeDtypeStruct((N, H), jnp.float32)],
    )(deg1p.reshape(2, NPAD, 1), h)

    Sp = _sc_spmm(row2d, col2d, ew2d, hp)

    wc = jnp.concatenate([W_out, W_root], axis=1)
    sr2 = pl.pallas_call(
        _tc_mid_body,
        out_shape=jax.ShapeDtypeStruct((N, 2), jnp.float32),
    )(Sp, hp, dis, b1.reshape(1, H), wc)

    s1 = sr2[:, 0]
    tp, d2p = _sc_layer2(row2d, col2d, s1)

    feat = pl.pallas_call(
        _tc_final_body,
        out_shape=jax.ShapeDtypeStruct((N, 1), jnp.float32),
    )(tp.reshape(2, NPAD, 1), d2p.reshape(2, NPAD, 1), sr2,
      b_out.reshape(1, 1))

    return feat.reshape(-1), feat


# speedup vs baseline: 40.7559x; 2.3984x over previous
import functools

import jax
import jax.numpy as jnp
from jax import lax
from jax.experimental import pallas as pl
from jax.experimental.pallas import tpu as pltpu
from jax.experimental.pallas import tpu_sc as plsc

N = 10000
E = 320000
D = 128
H = 128
NPAD = 10240
PART = NPAD // 16

NW = 32
WBLK = 80
EBP = NW * WBLK
EP = EBP * 128
NCH = WBLK // 8
BW = 64
WBLK2 = EP // NW // BW

_MESH = plsc.VectorSubcoreMesh(core_axis_name="c", subcore_axis_name="s")
_SC_PARAMS = pltpu.CompilerParams(needs_layout_passes=False)


def _worker_id():
    return lax.axis_index("s") * 2 + lax.axis_index("c")


def _splat(e):
    return jnp.full((16,), e, dtype=jnp.int32)


@functools.partial(
    pl.kernel,
    mesh=_MESH,
    compiler_params=_SC_PARAMS,
    out_type=jax.ShapeDtypeStruct((2, NPAD), jnp.float32),
    scratch_types=[
        pltpu.VMEM((3, 8, 128), jnp.int32),
        pltpu.VMEM((3, 8, 128), jnp.float32),
        pltpu.VMEM((PART,), jnp.float32),
        pltpu.VMEM_SHARED((NPAD,), jnp.float32),
        pltpu.SemaphoreType.DMA((3,)),
        pltpu.SemaphoreType.DMA((3,)),
    ],
)
def _sc_deg1(col_hbm, ew_hbm, out_hbm, cidx, ewb, zrow, acc, sem_i, sem_s):
    cid = lax.axis_index("c")
    sid = lax.axis_index("s")
    base = _worker_id() * WBLK

    def zbody(i, _):
        zrow[pl.ds(i * 16, 16)] = jnp.zeros((16,), jnp.float32)
        return 0
    lax.fori_loop(0, PART // 16, zbody, 0)
    pltpu.sync_copy(zrow, acc.at[pl.ds(sid * PART, PART)])
    plsc.subcore_barrier()

    def idx_descs(slot, c):
        blk = base + c * 8
        return (
            pltpu.make_async_copy(col_hbm.at[pl.ds(blk, 8)], cidx.at[slot],
                                  sem_i.at[slot]),
            pltpu.make_async_copy(ew_hbm.at[pl.ds(blk, 8)], ewb.at[slot],
                                  sem_i.at[slot]),
        )

    def issue_idx(slot, c):
        for d in idx_descs(slot, c):
            d.start()

    def wait_idx(slot):
        for d in idx_descs(slot, 0):
            d.wait()

    def scat_descs(slot):
        return [pltpu.make_async_copy(ewb.at[slot, k],
                                      acc.at[cidx.at[slot, k]],
                                      sem_s.at[slot]) for k in range(8)]

    def issue_scat(slot):
        for d in scat_descs(slot):
            d.start(add=True)

    def wait_scat(slot):
        for d in scat_descs(slot):
            d.wait()

    def step(t, slot, first):
        nxt = (slot + 2) % 3
        if not first:
            @pl.when(t < NCH - 2)
            def _():
                wait_scat(nxt)
                issue_idx(nxt, t + 2)
        else:
            issue_idx(nxt, t + 2)
        wait_idx(slot)
        issue_scat(slot)

    issue_idx(0, 0)
    issue_idx(1, 1)
    step(0, 0, True)

    def body(ti, _):
        t = 1 + ti * 3
        for off in range(3):
            step(t + off, (1 + off) % 3, False)
        return 0
    lax.fori_loop(0, (NCH - 1) // 3, body, 0)

    for slot in range(3):
        wait_scat(slot)
    plsc.subcore_barrier()
    pltpu.sync_copy(acc.at[pl.ds(sid * PART, PART)],
                    out_hbm.at[cid, pl.ds(sid * PART, PART)])


@functools.partial(
    pl.kernel,
    mesh=_MESH,
    compiler_params=_SC_PARAMS,
    out_type=jax.ShapeDtypeStruct((2, NPAD, 128), jnp.float32),
    scratch_types=[
        pltpu.VMEM((4, BW), jnp.int32),
        pltpu.VMEM((4, BW), jnp.int32),
        pltpu.VMEM((4, BW), jnp.float32),
        pltpu.VMEM((4, BW, 128), jnp.float32),
        pltpu.VMEM_SHARED((NPAD, 128), jnp.float32),
        pltpu.SemaphoreType.DMA((4,)),
        pltpu.SemaphoreType.DMA((4,)),
        pltpu.SemaphoreType.DMA((4,)),
    ],
)
def _sc_spmm(row_hbm, col_hbm, ew_hbm, hp_hbm, out_hbm,
             ridx, cidx, ewb, rows, acc, sem_i, sem_g, sem_s):
    cid = lax.axis_index("c")
    sid = lax.axis_index("s")
    base = _worker_id() * WBLK2

    def zbody(i, _):
        for j in range(8):
            rows[0, i, pl.ds(j * 16, 16)] = jnp.zeros((16,), jnp.float32)
        return 0
    lax.fori_loop(0, BW, zbody, 0)
    for k in range(PART // BW):
        pltpu.sync_copy(rows.at[0], acc.at[pl.ds(sid * PART + k * BW, BW)])
    plsc.subcore_barrier()

    def idx_descs(slot, t):
        blk = base + t
        return (
            pltpu.make_async_copy(row_hbm.at[blk], ridx.at[slot],
                                  sem_i.at[slot]),
            pltpu.make_async_copy(col_hbm.at[blk], cidx.at[slot],
                                  sem_i.at[slot]),
            pltpu.make_async_copy(ew_hbm.at[blk], ewb.at[slot],
                                  sem_i.at[slot]),
        )

    def issue_idx(slot, t):
        for d in idx_descs(slot, t):
            d.start()

    def wait_idx(slot):
        for d in idx_descs(slot, 0):
            d.wait()

    def gat_desc(slot):
        return pltpu.make_async_copy(hp_hbm.at[ridx.at[slot]],
                                     rows.at[slot], sem_g.at[slot])

    def scat_desc(slot):
        return pltpu.make_async_copy(rows.at[slot],
                                     acc.at[cidx.at[slot]], sem_s.at[slot])

    def scale(slot):
        def ebody(e, _):
            w16 = plsc.load_gather(ewb, [_splat(slot), _splat(e)])
            for j in range(8):
                sl = pl.ds(j * 16, 16)
                rows[slot, e, sl] = rows[slot, e, sl] * w16
            return 0
        lax.fori_loop(0, BW, ebody, 0)

    def step(t, slot, refill, gather_next, drain):
        if refill:
            nxt = (slot + 2) % 4
            if drain:
                wait_scat = scat_desc(nxt)
                wait_scat.wait()
            issue_idx(nxt, t + 2)
        if gather_next:
            ng = (slot + 1) % 4
            wait_idx(ng)
            gat_desc(ng).start()
        gat_desc(slot).wait()
        scale(slot)
        scat_desc(slot).start(add=True)

    issue_idx(0, 0)
    issue_idx(1, 1)
    wait_idx(0)
    gat_desc(0).start()

    step(0, 0, True, True, False)
    step(1, 1, True, True, False)

    def body(ti, _):
        t = 2 + ti * 4
        for off in range(4):
            step(t + off, (2 + off) % 4, True, True, True)
        return 0
    lax.fori_loop(0, (WBLK2 - 4) // 4, body, 0)

    step(WBLK2 - 2, (WBLK2 - 2) % 4, False, True, False)
    step(WBLK2 - 1, (WBLK2 - 1) % 4, False, False, False)

    for slot in range(4):
        scat_desc(slot).wait()
    plsc.subcore_barrier()
    pltpu.sync_copy(acc.at[pl.ds(sid * PART, PART)],
                    out_hbm.at[cid, pl.ds(sid * PART, PART)])


@functools.partial(
    pl.kernel,
    mesh=_MESH,
    compiler_params=_SC_PARAMS,
    out_type=[jax.ShapeDtypeStruct((2, NPAD), jnp.float32),
              jax.ShapeDtypeStruct((2, NPAD), jnp.float32)],
    scratch_types=[
        pltpu.VMEM((3, 8, 128), jnp.int32),
        pltpu.VMEM((3, 8, 128), jnp.int32),
        pltpu.VMEM((3, 8, 128), jnp.float32),
        pltpu.VMEM((3, 8, 128), jnp.float32),
        pltpu.VMEM((N,), jnp.float32),
        pltpu.VMEM((PART,), jnp.float32),
        pltpu.VMEM_SHARED((NPAD,), jnp.float32),
        pltpu.VMEM_SHARED((NPAD,), jnp.float32),
        pltpu.SemaphoreType.DMA((3,)),
        pltpu.SemaphoreType.DMA((3,)),
    ],
)
def _sc_layer2(row_hbm, col_hbm, s_hbm, tout_hbm, dout_hbm,
               ridx, cidx, updb, maskb, sv, zrow, tacc, dacc, sem_i, sem_s):
    cid = lax.axis_index("c")
    sid = lax.axis_index("s")
    base = _worker_id() * WBLK

    def zbody(i, _):
        zrow[pl.ds(i * 16, 16)] = jnp.zeros((16,), jnp.float32)
        return 0
    lax.fori_loop(0, PART // 16, zbody, 0)
    pltpu.sync_copy(zrow, tacc.at[pl.ds(sid * PART, PART)])
    pltpu.sync_copy(zrow, dacc.at[pl.ds(sid * PART, PART)])
    pltpu.sync_copy(s_hbm, sv)
    plsc.subcore_barrier()

    def idx_descs(slot, c):
        blk = base + c * 8
        return (
            pltpu.make_async_copy(row_hbm.at[pl.ds(blk, 8)], ridx.at[slot],
                                  sem_i.at[slot]),
            pltpu.make_async_copy(col_hbm.at[pl.ds(blk, 8)], cidx.at[slot],
                                  sem_i.at[slot]),
        )

    def issue_idx(slot, c):
        for d in idx_descs(slot, c):
            d.start()

    def wait_idx(slot):
        for d in idx_descs(slot, 0):
            d.wait()

    def scat_descs(slot):
        ds = []
        for k in range(8):
            ds.append(pltpu.make_async_copy(updb.at[slot, k],
                                            tacc.at[cidx.at[slot, k]],
                                            sem_s.at[slot]))
            ds.append(pltpu.make_async_copy(maskb.at[slot, k],
                                            dacc.at[cidx.at[slot, k]],
                                            sem_s.at[slot]))
        return ds

    def issue_scat(slot):
        for d in scat_descs(slot):
            d.start(add=True)

    def wait_scat(slot):
        for d in scat_descs(slot):
            d.wait()

    def compute(slot):
        for k in range(8):
            def gbody(g, _):
                sl = pl.ds(g * 16, 16)
                r16 = ridx[slot, k, sl]
                c16 = cidx[slot, k, sl]
                sval = plsc.load_gather(sv, [r16])
                m = r16 != c16
                updb[slot, k, sl] = jnp.where(m, sval, 0.0)
                maskb[slot, k, sl] = jnp.where(m, 1.0, 0.0)
                return 0
            lax.fori_loop(0, 8, gbody, 0)

    def step(t, slot, first):
        nxt = (slot + 2) % 3
        if not first:
            @pl.when(t < NCH - 2)
            def _():
                wait_scat(nxt)
                issue_idx(nxt, t + 2)
        else:
            issue_idx(nxt, t + 2)
        wait_idx(slot)
        compute(slot)
        issue_scat(slot)

    issue_idx(0, 0)
    issue_idx(1, 1)
    step(0, 0, True)

    def body(ti, _):
        t = 1 + ti * 3
        for off in range(3):
            step(t + off, (1 + off) % 3, False)
        return 0
    lax.fori_loop(0, (NCH - 1) // 3, body, 0)

    for slot in range(3):
        wait_scat(slot)
    plsc.subcore_barrier()
    pltpu.sync_copy(tacc.at[pl.ds(sid * PART, PART)],
                    tout_hbm.at[cid, pl.ds(sid * PART, PART)])
    pltpu.sync_copy(dacc.at[pl.ds(sid * PART, PART)],
                    dout_hbm.at[cid, pl.ds(sid * PART, PART)])


def _tc_prep_body(x_ref, w_ref, dp_ref, dis_ref, hp_ref):
    h = jnp.dot(x_ref[...], w_ref[...], preferred_element_type=jnp.float32)
    deg = dp_ref[0, :N, :] + dp_ref[1, :N, :] + 1.0
    dis = jnp.where(deg > 0, lax.rsqrt(jnp.maximum(deg, 1e-12)), 0.0)
    dis_ref[...] = dis
    hp_ref[...] = h * dis


def _tc_mid_body(S_ref, hp_ref, dis_ref, b1_ref, wc_ref, o_ref):
    pre = dis_ref[...] * (S_ref[0, :N, :] + S_ref[1, :N, :] + hp_ref[...]) \
        + b1_ref[...]
    hr = jnp.maximum(pre, 0.0)
    o_ref[...] = jnp.dot(hr, wc_ref[...], preferred_element_type=jnp.float32)


def _tc_final_body(tp_ref, dp_ref, sr_ref, bo_ref, o_ref):
    t = tp_ref[0, :N, :] + tp_ref[1, :N, :]
    d2 = jnp.maximum(dp_ref[0, :N, :] + dp_ref[1, :N, :] + 1.0, 1.0)
    s = sr_ref[:, 0:1]
    r = sr_ref[:, 1:2]
    o_ref[...] = (t + s) / d2 + bo_ref[0, 0] + r


def kernel(x, edge_index, edge_weight, W1, b1, W_out, b_out, W_root):
    npad_e = EP - E
    pad_idx = (jnp.arange(npad_e, dtype=jnp.int32) % N)
    row_p = jnp.concatenate([edge_index[0], pad_idx])
    col_p = jnp.concatenate([edge_index[1], pad_idx])
    ew_p = jnp.concatenate([edge_weight, jnp.zeros((npad_e,), jnp.float32)])
    row2d = row_p.reshape(EBP, 128)
    col2d = col_p.reshape(EBP, 128)
    ew2d = ew_p.reshape(EBP, 128)
    row2d64 = row_p.reshape(EP // BW, BW)
    col2d64 = col_p.reshape(EP // BW, BW)
    ew2d64 = ew_p.reshape(EP // BW, BW)

    deg1p = _sc_deg1(col2d, ew2d)

    dis, hp = pl.pallas_call(
        _tc_prep_body,
        out_shape=[jax.ShapeDtypeStruct((N, 1), jnp.float32),
                   jax.ShapeDtypeStruct((N, H), jnp.float32)],
    )(x, W1, deg1p.reshape(2, NPAD, 1))

    Sp = _sc_spmm(row2d64, col2d64, ew2d64, hp)

    wc = jnp.concatenate([W_out, W_root], axis=1)
    sr2 = pl.pallas_call(
        _tc_mid_body,
        out_shape=jax.ShapeDtypeStruct((N, 2), jnp.float32),
    )(Sp, hp, dis, b1.reshape(1, H), wc)

    s1 = sr2[:, 0]
    tp, d2p = _sc_layer2(row2d, col2d, s1)

    feat = pl.pallas_call(
        _tc_final_body,
        out_shape=jax.ShapeDtypeStruct((N, 1), jnp.float32),
    )(tp.reshape(2, NPAD, 1), d2p.reshape(2, NPAD, 1), sr2,
      b_out.reshape(1, 1))

    return feat.reshape(-1), feat
